# Initial kernel scaffold; baseline (speedup 1.0000x reference)
import jax
import jax.numpy as jnp
from jax.experimental import pallas as pl


def kernel(x, router_logits, w13, w2):
    raise NotImplementedError("write your pallas kernel here")


# speedup vs baseline: 1.1121x; 1.1121x over previous
import jax
import jax.numpy as jnp
from jax.experimental import pallas as pl
from jax.experimental.pallas import tpu as pltpu

_NUM_EXPERTS = 8
_TOP_K = 2
_HIDDEN = 1024
_INTER = 2816
_NUM_TOKENS = 32

_BI = 1408


def _moe_body(x_ref, rl_ref, w13_ref, w2_ref, out_ref, wte_ref):
    e = pl.program_id(0)
    i = pl.program_id(1)

    @pl.when((e == 0) & (i == 0))
    def _():
        logits = rl_ref[...]
        m = jnp.max(logits, axis=-1, keepdims=True)
        p = jnp.exp(logits - m)
        p = p / jnp.sum(p, axis=-1, keepdims=True)
        idx = jax.lax.broadcasted_iota(jnp.int32, p.shape, 1)
        m1 = jnp.max(p, axis=-1, keepdims=True)
        i1 = jnp.min(jnp.where(p == m1, idx, _NUM_EXPERTS), axis=-1,
                     keepdims=True)
        p2 = jnp.where(idx == i1, -jnp.inf, p)
        m2 = jnp.max(p2, axis=-1, keepdims=True)
        i2 = jnp.min(jnp.where(p2 == m2, idx, _NUM_EXPERTS), axis=-1,
                     keepdims=True)
        s = m1 + m2
        wte_ref[...] = jnp.where(
            idx == i1, m1, jnp.where(idx == i2, m2, 0.0)) / s
        out_ref[...] = jnp.zeros_like(out_ref)

    xb = x_ref[...].astype(jnp.bfloat16)
    gate_w = w13_ref[0, 0].astype(jnp.bfloat16)
    up_w = w13_ref[0, 1].astype(jnp.bfloat16)
    dims = (((1,), (1,)), ((), ()))
    gate = jax.lax.dot_general(xb, gate_w, dims,
                               preferred_element_type=jnp.float32)
    up = jax.lax.dot_general(xb, up_w, dims,
                             preferred_element_type=jnp.float32)
    act = gate * jax.nn.sigmoid(gate) * up

    eidx = jax.lax.broadcasted_iota(jnp.int32, (_NUM_TOKENS, _NUM_EXPERTS), 1)
    scale = jnp.sum(jnp.where(eidx == e, wte_ref[...], 0.0), axis=-1,
                    keepdims=True)
    actb = (act * scale).astype(jnp.bfloat16)
    w2b = w2_ref[0].astype(jnp.bfloat16)
    out_ref[...] += jax.lax.dot_general(
        actb, w2b, (((1,), (1,)), ((), ())),
        preferred_element_type=jnp.float32)


def kernel(x, router_logits, w13, w2):
    w13r = w13.reshape(_NUM_EXPERTS, 2, _INTER, _HIDDEN)
    grid = (_NUM_EXPERTS, _INTER // _BI)
    return pl.pallas_call(
        _moe_body,
        grid=grid,
        in_specs=[
            pl.BlockSpec((_NUM_TOKENS, _HIDDEN), lambda e, i: (0, 0)),
            pl.BlockSpec((_NUM_TOKENS, _NUM_EXPERTS), lambda e, i: (0, 0)),
            pl.BlockSpec((1, 2, _BI, _HIDDEN), lambda e, i: (e, 0, i, 0)),
            pl.BlockSpec((1, _HIDDEN, _BI), lambda e, i: (e, 0, i)),
        ],
        out_specs=pl.BlockSpec((_NUM_TOKENS, _HIDDEN), lambda e, i: (0, 0)),
        out_shape=jax.ShapeDtypeStruct((_NUM_TOKENS, _HIDDEN), jnp.float32),
        scratch_shapes=[pltpu.VMEM((_NUM_TOKENS, _NUM_EXPERTS), jnp.float32)],
        compiler_params=pltpu.CompilerParams(
            dimension_semantics=("arbitrary", "arbitrary")),
    )(x, router_logits, w13r, w2)
